# baseline (speedup 1.0000x reference)
import functools

import jax
import jax.numpy as jnp
from jax import lax
from jax.experimental import pallas as pl
from jax.experimental.pallas import tpu as pltpu
from jax.experimental.pallas import tpu_sc as plsc

ROWW = 80
_GDN = lax.GatherDimensionNumbers(
    offset_dims=(), collapsed_slice_dims=(0,), start_index_map=(0,))


def _xlane_sum(v):
    lane = lax.iota(jnp.int32, 16)
    for k in (1, 2, 4, 8):
        idx = jnp.bitwise_xor(lane, k)
        g = lax.gather(v, idx[:, None], _GDN, (1,),
                       mode=lax.GatherScatterMode.PROMISE_IN_BOUNDS)
        v = v + g
    return v
NC, NS = 2, 16
EB = 128


def _dual_mm(x, Wl, bl, Wr, br):
    N, F = x.shape
    H = Wl.shape[1]
    BN = 1000
    assert N % BN == 0

    def body(x_ref, wl_ref, bl_ref, wr_ref, br_ref, xl_ref, xr_ref):
        xb = x_ref[...]
        xl_ref[...] = (
            jnp.dot(xb, wl_ref[...], preferred_element_type=jnp.float32)
            + bl_ref[...]
        )
        xr_ref[...] = (
            jnp.dot(xb, wr_ref[...], preferred_element_type=jnp.float32)
            + br_ref[...]
        )

    return pl.pallas_call(
        body,
        grid=(N // BN,),
        in_specs=[
            pl.BlockSpec((BN, F), lambda i: (i, 0)),
            pl.BlockSpec((F, H), lambda i: (0, 0)),
            pl.BlockSpec((1, H), lambda i: (0, 0)),
            pl.BlockSpec((F, H), lambda i: (0, 0)),
            pl.BlockSpec((1, H), lambda i: (0, 0)),
        ],
        out_specs=[
            pl.BlockSpec((BN, H), lambda i: (i, 0)),
            pl.BlockSpec((BN, H), lambda i: (i, 0)),
        ],
        out_shape=[
            jax.ShapeDtypeStruct((N, H), jnp.float32),
            jax.ShapeDtypeStruct((N, H), jnp.float32),
        ],
    )(x, Wl, bl.reshape(1, H), Wr, br.reshape(1, H))


def _combine_relu_mm(acc, bias, Wl, bl, Wr, br, n_out):
    H = bias.shape[0]
    O = Wl.shape[1]
    BN = 1000
    assert n_out % BN == 0

    def body(a_ref, b1_ref, wl_ref, bl_ref, wr_ref, br_ref, xl_ref, xr_ref):
        a = a_ref[0] + a_ref[1]
        h = a[:, :H] / (a[:, H:H + 1] + 1e-16) + b1_ref[...]
        h = jnp.maximum(h, 0.0)
        xl_ref[...] = (
            jnp.dot(h, wl_ref[...], preferred_element_type=jnp.float32)
            + bl_ref[...]
        )
        xr_ref[...] = (
            jnp.dot(h, wr_ref[...], preferred_element_type=jnp.float32)
            + br_ref[...]
        )

    return pl.pallas_call(
        body,
        grid=(n_out // BN,),
        in_specs=[
            pl.BlockSpec((2, BN, ROWW), lambda i: (0, i, 0)),
            pl.BlockSpec((1, H), lambda i: (0, 0)),
            pl.BlockSpec((H, O), lambda i: (0, 0)),
            pl.BlockSpec((1, O), lambda i: (0, 0)),
            pl.BlockSpec((H, O), lambda i: (0, 0)),
            pl.BlockSpec((1, O), lambda i: (0, 0)),
        ],
        out_specs=[
            pl.BlockSpec((BN, O), lambda i: (i, 0)),
            pl.BlockSpec((BN, O), lambda i: (i, 0)),
        ],
        out_shape=[
            jax.ShapeDtypeStruct((n_out, O), jnp.float32),
            jax.ShapeDtypeStruct((n_out, O), jnp.float32),
        ],
    )(acc, bias.reshape(1, H), Wl, bl.reshape(1, O), Wr, br.reshape(1, O))


def _final_combine(acc, bias, n_out):
    O = bias.shape[0]
    BN = 1000
    assert n_out % BN == 0

    def body(a_ref, b_ref, o_ref):
        a = a_ref[0] + a_ref[1]
        o_ref[...] = a[:, :O] / (a[:, O:O + 1] + 1e-16) + b_ref[...]

    return pl.pallas_call(
        body,
        grid=(n_out // BN,),
        in_specs=[
            pl.BlockSpec((2, BN, ROWW), lambda i: (0, i, 0)),
            pl.BlockSpec((1, O), lambda i: (0, 0)),
        ],
        out_specs=pl.BlockSpec((BN, O), lambda i: (i, 0)),
        out_shape=jax.ShapeDtypeStruct((n_out, O), jnp.float32),
    )(acc, bias.reshape(1, O))


def _edge_pass(xl, xr, att, edge_index, npad):
    E = edge_index.shape[1]
    H = xl.shape[1]
    assert H == 64 and E % EB == 0
    nch = E // EB
    nw = NC * NS
    per_w = -(-nch // nw)
    rows_t = npad // NS
    assert rows_t % EB == 0

    mesh = plsc.VectorSubcoreMesh(
        core_axis_name="c", subcore_axis_name="s",
        num_cores=NC, num_subcores=NS,
    )

    @functools.partial(
        pl.kernel,
        out_type=jax.ShapeDtypeStruct((NC, npad, ROWW), jnp.float32),
        mesh=mesh,
        compiler_params=pltpu.CompilerParams(use_tc_tiling_on_sc=False),
        scratch_types=(
            [pltpu.VMEM((EB,), jnp.int32)] * 2
            + [pltpu.VMEM((EB,), jnp.int32)] * 2
            + [pltpu.VMEM((EB,), jnp.int32)] * 2
            + [pltpu.VMEM((EB, 64), jnp.float32)] * 2
            + [pltpu.VMEM((EB, 64), jnp.float32)] * 2
            + [pltpu.VMEM((EB, ROWW), jnp.float32)] * 2
            + [pltpu.VMEM((H,), jnp.float32)]
            + [pltpu.VMEM_SHARED((npad, ROWW), jnp.float32)]
            + [pltpu.SemaphoreType.DMA] * 6
        ),
    )
    def k(xl_hbm, xr_hbm, att_hbm, edge_hbm, out_hbm,
          si0, si1, di0, di1, sdi0, sdi1, gl0, gl1, gr0, gr1,
          ob0, ob1, attv, acc,
          gs0, gs1, gs2, gs3, ss0, ss1):
        cid = lax.axis_index("c")
        sid = lax.axis_index("s")
        wid = sid * NC + cid
        bufs = [(si0, di0, gl0, gr0, gs0, gs1, ob0, ss0, sdi0),
                (si1, di1, gl1, gr1, gs2, gs3, ob1, ss1, sdi1)]

        def fetch(c, bu):
            @pl.when(c < nch)
            def _():
                base = c * EB
                pltpu.sync_copy(edge_hbm.at[0, pl.ds(base, EB)], bu[0])
                pltpu.sync_copy(edge_hbm.at[1, pl.ds(base, EB)], bu[1])
                pltpu.async_copy(xl_hbm.at[bu[0]], bu[2], bu[4])
                pltpu.async_copy(xr_hbm.at[bu[1]], bu[3], bu[5])

        pltpu.sync_copy(att_hbm, attv)
        fetch(wid, bufs[0])

        def zrow(i, _):
            for c4 in range(ROWW // 16):
                ob0[i, pl.ds(c4 * 16, 16)] = jnp.zeros((16,), jnp.float32)
            return _
        lax.fori_loop(0, EB, zrow, None)

        def zacc(r, _):
            pltpu.sync_copy(ob0, acc.at[pl.ds(sid * rows_t + r * EB, EB)])
            return _
        lax.fori_loop(0, rows_t // EB, zacc, None)
        plsc.subcore_barrier()

        def chunk_body(kk, _):
            for b in range(2):
                i = kk * 2 + b
                cidx = wid + i * nw
                bu = bufs[b]
                nbu = bufs[1 - b]

                @pl.when(cidx < nch)
                def _(cidx=cidx, bu=bu):
                    pltpu.make_async_copy(xl_hbm.at[bu[0]], bu[2], bu[4]).wait()
                    pltpu.make_async_copy(xr_hbm.at[bu[1]], bu[3], bu[5]).wait()

                @pl.when(jnp.logical_and(i >= 2, cidx - 2 * nw < nch))
                def _(cidx=cidx, bu=bu):
                    pltpu.make_async_copy(
                        bu[6], acc.at[bu[8]], bu[7]).wait()

                fetch(cidx + nw, nbu)

                @pl.when(cidx < nch)
                def _(cidx=cidx, bu=bu):
                    gl, gr, ob = bu[2], bu[3], bu[6]
                    for c8 in range(EB // 16):
                        bu[8][pl.ds(c8 * 16, 16)] = bu[1][pl.ds(c8 * 16, 16)]
                    atts = [attv[pl.ds(c4 * 16, 16)] for c4 in range(H // 16)]
                    lane0 = lax.iota(jnp.int32, 16) == 0

                    @plsc.parallel_loop(0, EB, 1, unroll=8)
                    def _edges(e):
                        vls = []
                        accv = jnp.zeros((16,), jnp.float32)
                        for c4 in range(H // 16):
                            vl = gl[e, pl.ds(c4 * 16, 16)]
                            vr = gr[e, pl.ds(c4 * 16, 16)]
                            v = vl + vr
                            v = jnp.where(v >= 0.0, v, v * 0.2)
                            accv = accv + v * atts[c4]
                            vls.append(vl)
                        pv = jnp.exp(_xlane_sum(accv))
                        for c4 in range(H // 16):
                            ob[e, pl.ds(c4 * 16, 16)] = pv * vls[c4]
                        ob[e, pl.ds(H, 16)] = jnp.where(lane0, pv, 0.0)

                    pltpu.async_copy(ob, acc.at[bu[8]], bu[7], add=True)
            return _
        lax.fori_loop(0, (per_w + 2 + 1) // 2, chunk_body, None)

        plsc.subcore_barrier()
        pltpu.sync_copy(acc.at[pl.ds(sid * rows_t, rows_t)],
                        out_hbm.at[cid, pl.ds(sid * rows_t, rows_t)])

    return k(xl, xr, att, edge_index)


def kernel(inputs, edge_index, W1l, b1l, W1r, b1r, att1, bias1,
           W2l, b2l, W2r, b2r, att2, bias2):
    N = inputs.shape[0]
    npad = ((N + NS * EB - 1) // (NS * EB)) * (NS * EB)

    xl1, xr1 = _dual_mm(inputs, W1l, b1l, W1r, b1r)
    acc1 = _edge_pass(xl1, xr1, att1, edge_index, npad)
    xl2, xr2 = _combine_relu_mm(acc1, bias1, W2l, b2l, W2r, b2r, N)
    acc2 = _edge_pass(xl2, xr2, att2, edge_index, npad)
    return _final_combine(acc2, bias2, N)


# speedup vs baseline: 1.1019x; 1.1019x over previous
import functools

import jax
import jax.numpy as jnp
from jax import lax
from jax.experimental import pallas as pl
from jax.experimental.pallas import tpu as pltpu
from jax.experimental.pallas import tpu_sc as plsc

ROWW = 72
_GDN = lax.GatherDimensionNumbers(
    offset_dims=(), collapsed_slice_dims=(0,), start_index_map=(0,))


def _xlane_sum(v):
    lane = lax.iota(jnp.int32, 16)
    for k in (1, 2, 4, 8):
        idx = jnp.bitwise_xor(lane, k)
        g = lax.gather(v, idx[:, None], _GDN, (1,),
                       mode=lax.GatherScatterMode.PROMISE_IN_BOUNDS)
        v = v + g
    return v
NC, NS = 2, 16
EB = 128


def _dual_mm(x, Wl, bl, Wr, br):
    N, F = x.shape
    H = Wl.shape[1]
    BN = 1000
    assert N % BN == 0

    def body(x_ref, wl_ref, bl_ref, wr_ref, br_ref, xl_ref, xr_ref):
        xb = x_ref[...]
        xl_ref[...] = (
            jnp.dot(xb, wl_ref[...], preferred_element_type=jnp.float32)
            + bl_ref[...]
        )
        xr_ref[...] = (
            jnp.dot(xb, wr_ref[...], preferred_element_type=jnp.float32)
            + br_ref[...]
        )

    return pl.pallas_call(
        body,
        grid=(N // BN,),
        in_specs=[
            pl.BlockSpec((BN, F), lambda i: (i, 0)),
            pl.BlockSpec((F, H), lambda i: (0, 0)),
            pl.BlockSpec((1, H), lambda i: (0, 0)),
            pl.BlockSpec((F, H), lambda i: (0, 0)),
            pl.BlockSpec((1, H), lambda i: (0, 0)),
        ],
        out_specs=[
            pl.BlockSpec((BN, H), lambda i: (i, 0)),
            pl.BlockSpec((BN, H), lambda i: (i, 0)),
        ],
        out_shape=[
            jax.ShapeDtypeStruct((N, H), jnp.float32),
            jax.ShapeDtypeStruct((N, H), jnp.float32),
        ],
    )(x, Wl, bl.reshape(1, H), Wr, br.reshape(1, H))


def _combine_relu_mm(acc, bias, Wl, bl, Wr, br, n_out):
    H = bias.shape[0]
    O = Wl.shape[1]
    BN = 1000
    assert n_out % BN == 0

    def body(a_ref, b1_ref, wl_ref, bl_ref, wr_ref, br_ref, xl_ref, xr_ref):
        a = a_ref[0] + a_ref[1]
        h = a[:, :H] / (a[:, H:H + 1] + 1e-16) + b1_ref[...]
        h = jnp.maximum(h, 0.0)
        xl_ref[...] = (
            jnp.dot(h, wl_ref[...], preferred_element_type=jnp.float32)
            + bl_ref[...]
        )
        xr_ref[...] = (
            jnp.dot(h, wr_ref[...], preferred_element_type=jnp.float32)
            + br_ref[...]
        )

    return pl.pallas_call(
        body,
        grid=(n_out // BN,),
        in_specs=[
            pl.BlockSpec((2, BN, ROWW), lambda i: (0, i, 0)),
            pl.BlockSpec((1, H), lambda i: (0, 0)),
            pl.BlockSpec((H, O), lambda i: (0, 0)),
            pl.BlockSpec((1, O), lambda i: (0, 0)),
            pl.BlockSpec((H, O), lambda i: (0, 0)),
            pl.BlockSpec((1, O), lambda i: (0, 0)),
        ],
        out_specs=[
            pl.BlockSpec((BN, O), lambda i: (i, 0)),
            pl.BlockSpec((BN, O), lambda i: (i, 0)),
        ],
        out_shape=[
            jax.ShapeDtypeStruct((n_out, O), jnp.float32),
            jax.ShapeDtypeStruct((n_out, O), jnp.float32),
        ],
    )(acc, bias.reshape(1, H), Wl, bl.reshape(1, O), Wr, br.reshape(1, O))


def _final_combine(acc, bias, n_out):
    O = bias.shape[0]
    BN = 1000
    assert n_out % BN == 0

    def body(a_ref, b_ref, o_ref):
        a = a_ref[0] + a_ref[1]
        o_ref[...] = a[:, :O] / (a[:, O:O + 1] + 1e-16) + b_ref[...]

    return pl.pallas_call(
        body,
        grid=(n_out // BN,),
        in_specs=[
            pl.BlockSpec((2, BN, ROWW), lambda i: (0, i, 0)),
            pl.BlockSpec((1, O), lambda i: (0, 0)),
        ],
        out_specs=pl.BlockSpec((BN, O), lambda i: (i, 0)),
        out_shape=jax.ShapeDtypeStruct((n_out, O), jnp.float32),
    )(acc, bias.reshape(1, O))


def _edge_pass(xl, xr, att, edge_index, npad):
    E = edge_index.shape[1]
    H = xl.shape[1]
    assert H == 64 and E % EB == 0
    nch = E // EB
    nw = NC * NS
    per_w = -(-nch // nw)
    rows_t = npad // NS
    assert rows_t % EB == 0

    mesh = plsc.VectorSubcoreMesh(
        core_axis_name="c", subcore_axis_name="s",
        num_cores=NC, num_subcores=NS,
    )

    @functools.partial(
        pl.kernel,
        out_type=jax.ShapeDtypeStruct((NC, npad, ROWW), jnp.float32),
        mesh=mesh,
        compiler_params=pltpu.CompilerParams(use_tc_tiling_on_sc=False),
        scratch_types=(
            [pltpu.VMEM((EB,), jnp.int32)] * 3
            + [pltpu.VMEM((EB,), jnp.int32)] * 3
            + [pltpu.VMEM((EB,), jnp.int32)] * 3
            + [pltpu.VMEM((EB, 64), jnp.float32)] * 3
            + [pltpu.VMEM((EB, 64), jnp.float32)] * 3
            + [pltpu.VMEM((EB, ROWW), jnp.float32)] * 3
            + [pltpu.VMEM((H,), jnp.float32)]
            + [pltpu.VMEM_SHARED((npad, ROWW), jnp.float32)]
            + [pltpu.SemaphoreType.DMA] * 9
        ),
    )
    def k(xl_hbm, xr_hbm, att_hbm, edge_hbm, out_hbm,
          si0, si1, si2, di0, di1, di2, sdi0, sdi1, sdi2,
          gl0, gl1, gl2, gr0, gr1, gr2, ob0, ob1, ob2, attv, acc,
          gs0, gs1, gs2, gs3, gs4, gs5, ss0, ss1, ss2):
        cid = lax.axis_index("c")
        sid = lax.axis_index("s")
        wid = sid * NC + cid
        bufs = [(si0, di0, gl0, gr0, gs0, gs1, ob0, ss0, sdi0),
                (si1, di1, gl1, gr1, gs2, gs3, ob1, ss1, sdi1),
                (si2, di2, gl2, gr2, gs4, gs5, ob2, ss2, sdi2)]

        def fetch(c, bu):
            @pl.when(c < nch)
            def _():
                base = c * EB
                pltpu.sync_copy(edge_hbm.at[0, pl.ds(base, EB)], bu[0])
                pltpu.sync_copy(edge_hbm.at[1, pl.ds(base, EB)], bu[1])
                pltpu.async_copy(xl_hbm.at[bu[0]], bu[2], bu[4])
                pltpu.async_copy(xr_hbm.at[bu[1]], bu[3], bu[5])

        pltpu.sync_copy(att_hbm, attv)
        fetch(wid, bufs[0])
        fetch(wid + nw, bufs[1])

        def zrow(i, _):
            for off in (0, 16, 32, 48, 56):
                ob0[i, pl.ds(off, 16)] = jnp.zeros((16,), jnp.float32)
            return _
        lax.fori_loop(0, EB, zrow, None)

        def zacc(r, _):
            pltpu.sync_copy(ob0, acc.at[pl.ds(sid * rows_t + r * EB, EB)])
            return _
        lax.fori_loop(0, rows_t // EB, zacc, None)
        plsc.subcore_barrier()

        def chunk_body(kk, _):
            for b in range(3):
                i = kk * 3 + b
                cidx = wid + i * nw
                bu = bufs[b]
                nbu = bufs[(b + 2) % 3]

                @pl.when(cidx < nch)
                def _(cidx=cidx, bu=bu):
                    pltpu.make_async_copy(xl_hbm.at[bu[0]], bu[2], bu[4]).wait()
                    pltpu.make_async_copy(xr_hbm.at[bu[1]], bu[3], bu[5]).wait()

                @pl.when(jnp.logical_and(i >= 3, cidx - 3 * nw < nch))
                def _(cidx=cidx, bu=bu):
                    pltpu.make_async_copy(
                        bu[6], acc.at[bu[8]], bu[7]).wait()

                fetch(cidx + 2 * nw, nbu)

                @pl.when(cidx < nch)
                def _(cidx=cidx, bu=bu):
                    gl, gr, ob = bu[2], bu[3], bu[6]
                    for c8 in range(EB // 16):
                        bu[8][pl.ds(c8 * 16, 16)] = bu[1][pl.ds(c8 * 16, 16)]
                    atts = [attv[pl.ds(c4 * 16, 16)] for c4 in range(H // 16)]
                    lane = lax.iota(jnp.int32, 16)
                    lo8 = lane < 8
                    lane8 = lane == 8
                    xor8 = jnp.bitwise_xor(lane, 8)

                    @plsc.parallel_loop(0, EB, 1, unroll=4)
                    def _edges(e):
                        vls = []
                        accv = jnp.zeros((16,), jnp.float32)
                        for c4 in range(H // 16):
                            vl = gl[e, pl.ds(c4 * 16, 16)]
                            vr = gr[e, pl.ds(c4 * 16, 16)]
                            v = vl + vr
                            v = jnp.where(v >= 0.0, v, v * 0.2)
                            accv = accv + v * atts[c4]
                            vls.append(vl)
                        pv = jnp.exp(_xlane_sum(accv))
                        for c4 in range(3):
                            ob[e, pl.ds(c4 * 16, 16)] = pv * vls[c4]
                        t3 = pv * vls[3]
                        ob[e, pl.ds(48, 16)] = t3
                        hi = lax.gather(t3, xor8[:, None], _GDN, (1,),
                                        mode=lax.GatherScatterMode.PROMISE_IN_BOUNDS)
                        tail = jnp.where(lo8, hi, jnp.where(lane8, pv, 0.0))
                        ob[e, pl.ds(56, 16)] = tail

                    pltpu.async_copy(ob, acc.at[bu[8]], bu[7], add=True)
            return _
        lax.fori_loop(0, (per_w + 5) // 3, chunk_body, None)

        plsc.subcore_barrier()
        pltpu.sync_copy(acc.at[pl.ds(sid * rows_t, rows_t)],
                        out_hbm.at[cid, pl.ds(sid * rows_t, rows_t)])

    return k(xl, xr, att, edge_index)


def kernel(inputs, edge_index, W1l, b1l, W1r, b1r, att1, bias1,
           W2l, b2l, W2r, b2r, att2, bias2):
    N = inputs.shape[0]
    npad = ((N + NS * EB - 1) // (NS * EB)) * (NS * EB)

    xl1, xr1 = _dual_mm(inputs, W1l, b1l, W1r, b1r)
    acc1 = _edge_pass(xl1, xr1, att1, edge_index, npad)
    xl2, xr2 = _combine_relu_mm(acc1, bias1, W2l, b2l, W2r, b2r, N)
    acc2 = _edge_pass(xl2, xr2, att2, edge_index, npad)
    return _final_combine(acc2, bias2, N)
